# baseline (speedup 1.0000x reference)
import functools

import jax
import jax.numpy as jnp
from jax import lax
from jax.experimental import pallas as pl
from jax.experimental.pallas import tpu as pltpu
from jax.experimental.pallas import tpu_sc as plsc

B = 4096
D = 128
K = 8

NC = 2
NS = 16
NW = NC * NS
BPW = B // NW

BM = 512


_sc_mesh = plsc.VectorSubcoreMesh(core_axis_name="c", subcore_axis_name="s")


@functools.partial(
    pl.kernel,
    mesh=_sc_mesh,
    out_type=(
        jax.ShapeDtypeStruct((B, D), jnp.float32),
        jax.ShapeDtypeStruct((B, D), jnp.float32),
    ),
    scratch_types=[
        pltpu.VMEM((BPW,), jnp.int32),
        pltpu.VMEM((BPW,), jnp.int32),
        pltpu.VMEM((BPW, D), jnp.float32),
        pltpu.VMEM((BPW, D), jnp.float32),
        pltpu.SemaphoreType.DMA,
        pltpu.SemaphoreType.DMA,
    ],
)
def _sc_gather(uid_hbm, iid_hbm, utab_hbm, itab_hbm, uout_hbm, iout_hbm,
               uidx_v, iidx_v, urows_v, irows_v, usem, isem):
    wid = lax.axis_index("s") * NC + lax.axis_index("c")
    base = wid * BPW
    pltpu.sync_copy(uid_hbm.at[pl.ds(base, BPW)], uidx_v)
    pltpu.sync_copy(iid_hbm.at[pl.ds(base, BPW)], iidx_v)
    cu = pltpu.async_copy(utab_hbm.at[uidx_v], urows_v, usem)
    ci = pltpu.async_copy(itab_hbm.at[iidx_v], irows_v, isem)
    cu.wait()
    ci.wait()
    pltpu.sync_copy(urows_v, uout_hbm.at[pl.ds(base, BPW)])
    pltpu.sync_copy(irows_v, iout_hbm.at[pl.ds(base, BPW)])


def _tc_body(uemb_ref, iemb_ref, w_ref, b_ref, iv_ref, out_ref, ui_scr):
    @pl.when(pl.program_id(0) == 0)
    def _():
        logits = jnp.dot(uemb_ref[...], w_ref[...],
                         preferred_element_type=jnp.float32) + b_ref[...]
        m = jnp.max(logits, axis=-1, keepdims=True)
        e = jnp.exp(logits - m)
        scores = e / jnp.sum(e, axis=-1, keepdims=True)
        ui = jnp.dot(scores, iv_ref[...], preferred_element_type=jnp.float32)
        ui_scr[...] = ui.astype(jnp.bfloat16)

    sims = lax.dot_general(
        ui_scr[...], iemb_ref[...].astype(jnp.bfloat16),
        (((1,), (1,)), ((), ())),
        preferred_element_type=jnp.float32)
    out_ref[...] = jnp.max(sims, axis=0)


def kernel(user_ids, item_ids, user_table, item_table, interest_vectors,
           attn_W, attn_b):
    uids = user_ids.astype(jnp.int32)
    iids = item_ids.astype(jnp.int32)
    user_emb, item_emb = _sc_gather(uids, iids, user_table, item_table)
    return pl.pallas_call(
        _tc_body,
        grid=(B // BM,),
        in_specs=[
            pl.BlockSpec((B, D), lambda i: (0, 0)),
            pl.BlockSpec((BM, D), lambda i: (i, 0)),
            pl.BlockSpec((D, K), lambda i: (0, 0)),
            pl.BlockSpec((1, K), lambda i: (0, 0)),
            pl.BlockSpec((K, D), lambda i: (0, 0)),
        ],
        out_specs=pl.BlockSpec((BM,), lambda i: (i,)),
        scratch_shapes=[pltpu.VMEM((B, D), jnp.bfloat16)],
        out_shape=jax.ShapeDtypeStruct((B,), jnp.float32),
    )(user_emb, item_emb, attn_W, attn_b.reshape(1, K), interest_vectors)


# speedup vs baseline: 1.0381x; 1.0381x over previous
import functools

import jax
import jax.numpy as jnp
from jax import lax
from jax.experimental import pallas as pl
from jax.experimental.pallas import tpu as pltpu
from jax.experimental.pallas import tpu_sc as plsc

B = 4096
D = 128
K = 8

NC = 2
NS = 16
NW = NC * NS
BPW = B // NW

BM = 512


_sc_mesh = plsc.VectorSubcoreMesh(core_axis_name="c", subcore_axis_name="s")


@functools.partial(
    pl.kernel,
    mesh=_sc_mesh,
    out_type=(
        jax.ShapeDtypeStruct((B, D), jnp.float32),
        jax.ShapeDtypeStruct((B, D), jnp.float32),
    ),
    scratch_types=[
        pltpu.VMEM((BPW,), jnp.int32),
        pltpu.VMEM((BPW,), jnp.int32),
        pltpu.VMEM((BPW, D), jnp.float32),
        pltpu.VMEM((BPW, D), jnp.float32),
        pltpu.SemaphoreType.DMA,
        pltpu.SemaphoreType.DMA,
    ],
)
def _sc_gather(uid_hbm, iid_hbm, utab_hbm, itab_hbm, uout_hbm, iout_hbm,
               uidx_v, iidx_v, urows_v, irows_v, usem, isem):
    wid = lax.axis_index("s") * NC + lax.axis_index("c")
    base = wid * BPW
    pltpu.sync_copy(uid_hbm.at[pl.ds(base, BPW)], uidx_v)
    pltpu.sync_copy(iid_hbm.at[pl.ds(base, BPW)], iidx_v)
    cu = pltpu.async_copy(utab_hbm.at[uidx_v], urows_v, usem)
    ci = pltpu.async_copy(itab_hbm.at[iidx_v], irows_v, isem)
    cu.wait()
    ci.wait()
    pltpu.sync_copy(urows_v, uout_hbm.at[pl.ds(base, BPW)])
    pltpu.sync_copy(irows_v, iout_hbm.at[pl.ds(base, BPW)])


def _tc_body(uemb_ref, iemb_ref, w_ref, b_ref, iv_ref, out_ref):
    logits = jnp.dot(uemb_ref[...], w_ref[...],
                     preferred_element_type=jnp.float32) + b_ref[...]
    m = jnp.max(logits, axis=-1, keepdims=True)
    e = jnp.exp(logits - m)
    scores = e / jnp.sum(e, axis=-1, keepdims=True)
    ui = jnp.dot(scores, iv_ref[...], preferred_element_type=jnp.float32)
    ui_bf = ui.astype(jnp.bfloat16)
    for i in range(B // BM):
        sims = lax.dot_general(
            ui_bf, iemb_ref[pl.ds(i * BM, BM), :].astype(jnp.bfloat16),
            (((1,), (1,)), ((), ())),
            preferred_element_type=jnp.float32)
        out_ref[pl.ds(i * BM, BM)] = jnp.max(sims, axis=0)


def kernel(user_ids, item_ids, user_table, item_table, interest_vectors,
           attn_W, attn_b):
    uids = user_ids.astype(jnp.int32)
    iids = item_ids.astype(jnp.int32)
    user_emb, item_emb = _sc_gather(uids, iids, user_table, item_table)
    return pl.pallas_call(
        _tc_body,
        out_shape=jax.ShapeDtypeStruct((B,), jnp.float32),
    )(user_emb, item_emb, attn_W, attn_b.reshape(1, K), interest_vectors)


# speedup vs baseline: 1.0567x; 1.0180x over previous
import functools

import jax
import jax.numpy as jnp
from jax import lax
from jax.experimental import pallas as pl
from jax.experimental.pallas import tpu as pltpu
from jax.experimental.pallas import tpu_sc as plsc

B = 4096
D = 128
K = 8

NC = 2
NS = 16
NW = NC * NS
BPW = B // NW

BM = 512


_sc_mesh = plsc.VectorSubcoreMesh(core_axis_name="c", subcore_axis_name="s")


@functools.partial(
    pl.kernel,
    mesh=_sc_mesh,
    out_type=(
        jax.ShapeDtypeStruct((B, D), jnp.float32),
        jax.ShapeDtypeStruct((B, D), jnp.float32),
    ),
    scratch_types=[
        pltpu.VMEM((BPW,), jnp.int32),
        pltpu.VMEM((BPW,), jnp.int32),
        pltpu.VMEM((BPW, D), jnp.float32),
        pltpu.VMEM((BPW, D), jnp.float32),
        pltpu.SemaphoreType.DMA,
        pltpu.SemaphoreType.DMA,
        pltpu.SemaphoreType.DMA,
        pltpu.SemaphoreType.DMA,
    ],
)
def _sc_gather(uid_hbm, iid_hbm, utab_hbm, itab_hbm, uout_hbm, iout_hbm,
               uidx_v, iidx_v, urows_v, irows_v, usem, isem, wusem, wisem):
    wid = lax.axis_index("s") * NC + lax.axis_index("c")
    base = wid * BPW
    lu = pltpu.async_copy(uid_hbm.at[pl.ds(base, BPW)], uidx_v, usem)
    li = pltpu.async_copy(iid_hbm.at[pl.ds(base, BPW)], iidx_v, isem)
    lu.wait()
    cu = pltpu.async_copy(utab_hbm.at[uidx_v], urows_v, usem)
    li.wait()
    ci = pltpu.async_copy(itab_hbm.at[iidx_v], irows_v, isem)
    cu.wait()
    wu = pltpu.async_copy(urows_v, uout_hbm.at[pl.ds(base, BPW)], wusem)
    ci.wait()
    wi = pltpu.async_copy(irows_v, iout_hbm.at[pl.ds(base, BPW)], wisem)
    wu.wait()
    wi.wait()


def _tc_body(uemb_ref, iemb_ref, w_ref, b_ref, iv_ref, out_ref):
    logits = jnp.dot(uemb_ref[...], w_ref[...],
                     preferred_element_type=jnp.float32) + b_ref[...]
    m = jnp.max(logits, axis=-1, keepdims=True)
    e = jnp.exp(logits - m)
    scores = e / jnp.sum(e, axis=-1, keepdims=True)
    ui = jnp.dot(scores, iv_ref[...], preferred_element_type=jnp.float32)
    ui_bf = ui.astype(jnp.bfloat16)
    for i in range(B // BM):
        sims = lax.dot_general(
            ui_bf, iemb_ref[pl.ds(i * BM, BM), :].astype(jnp.bfloat16),
            (((1,), (1,)), ((), ())),
            preferred_element_type=jnp.float32)
        out_ref[pl.ds(i * BM, BM)] = jnp.max(sims, axis=0)


def kernel(user_ids, item_ids, user_table, item_table, interest_vectors,
           attn_W, attn_b):
    uids = user_ids.astype(jnp.int32)
    iids = item_ids.astype(jnp.int32)
    user_emb, item_emb = _sc_gather(uids, iids, user_table, item_table)
    return pl.pallas_call(
        _tc_body,
        out_shape=jax.ShapeDtypeStruct((B,), jnp.float32),
    )(user_emb, item_emb, attn_W, attn_b.reshape(1, K), interest_vectors)
